# baseline (speedup 1.0000x reference)
import functools

import jax
import jax.numpy as jnp
from jax import lax
from jax.experimental import pallas as pl
from jax.experimental.pallas import tpu as pltpu
from jax.experimental.pallas import tpu_sc as plsc

_B = 16384
_D = 64
_V = 100000
_NC = 2
_NS = 16
_NW = _NC * _NS
_BPW = _B // _NW
_C = 64
_NCHUNK = _BPW // _C
_NT = 4
_L = 16

_RB = 4096
_RBH = _RB // 2
_RB_SHIFT = 12
_NRB = (_V + _RB - 1) // _RB
_VP = _NRB * _RBH
_TC_BLK = 4096

_INV_SQRT_2PI = 0.3989422804014327


def _repack_body(t_ref, o_ref):
    eye = jnp.eye(_D, dtype=jnp.float32)
    t = lax.dot_general(t_ref[...], eye, (((0,), (0,)), ((), ())),
                        preferred_element_type=jnp.float32)
    o_ref[...] = jnp.concatenate([t[:_RBH, :], t[_RBH:, :]], axis=1)


def _gather_body(idx_hbm, t_hbm, o_hbm,
                 idx_v, pidx_v0, pidx_v1, g_v, gsems, wsems):
    wid = lax.axis_index("s") * _NC + lax.axis_index("c")
    wbase = wid * _BPW
    pidx_bufs = (pidx_v0, pidx_v1)

    def start_gather(c, buf):
        base = wbase + c * _C
        pltpu.sync_copy(idx_hbm.at[pl.ds(base, _C)], idx_v)
        pidx = pidx_bufs[buf]
        for j in range(_C // _L):
            sl = pl.ds(j * _L, _L)
            r = idx_v[sl]
            pidx[sl] = lax.shift_left(
                lax.shift_right_logical(r, _RB_SHIFT), _RB_SHIFT) \
                | lax.shift_left(r & (_RBH - 1), 1) \
                | (lax.shift_right_logical(r, _RB_SHIFT - 1) & 1)
        return pltpu.async_copy(t_hbm.at[pidx], g_v.at[buf], gsems.at[buf])

    def start_write(c, buf):
        base = wbase + c * _C
        return pltpu.async_copy(g_v.at[buf], o_hbm.at[pl.ds(base, _C)],
                                wsems.at[buf])

    pending = [None, None]
    dma = start_gather(0, 0)
    for c in range(_NCHUNK):
        buf = c % 2
        nxt = None
        if c + 1 < _NCHUNK:
            nbuf = (c + 1) % 2
            if pending[nbuf] is not None:
                pending[nbuf].wait()
                pending[nbuf] = None
            nxt = start_gather(c + 1, nbuf)
        dma.wait()
        pending[buf] = start_write(c, buf)
        dma = nxt
    for pw in pending:
        if pw is not None:
            pw.wait()


def _pdf_body(x_ref, lo_ref, p_ref, sp_ref, sb_ref, o_ref):
    x = x_ref[...]
    lo = lo_ref[...]
    p = p_ref[...]
    sp = sp_ref[...]
    sb = sb_ref[...]
    diff = x - lo
    isp = 1.0 / sp
    isb = 1.0 / sb
    zs = diff * isp
    zb = diff * isb
    es = jnp.exp(-0.5 * (zs * zs))
    eb = jnp.exp(-0.5 * (zb * zb))
    o_ref[...] = (p * _INV_SQRT_2PI) * isp * es + \
                 ((1.0 - p) * _INV_SQRT_2PI) * isb * eb


@jax.jit
def _spike_slab(X, indices, loc, pi, spike, slab):
    tspec = pl.BlockSpec((_D, _RB), lambda i: (0, i))
    pspec = pl.BlockSpec((_RB // 2, 2 * _D), lambda i: (i, 0))
    repack = pl.pallas_call(
        _repack_body,
        grid=(_NRB,),
        in_specs=[tspec],
        out_specs=pspec,
        out_shape=jax.ShapeDtypeStruct((_VP, 2 * _D), jnp.float32),
    )

    mesh = plsc.VectorSubcoreMesh(core_axis_name="c", subcore_axis_name="s",
                                  num_cores=_NC, num_subcores=_NS)
    gather = pl.kernel(
        _gather_body,
        out_type=jax.ShapeDtypeStruct((_B, _D), jnp.float32),
        mesh=mesh,
        scratch_types=[
            pltpu.VMEM((_C,), jnp.int32),
            pltpu.VMEM((_C,), jnp.int32),
            pltpu.VMEM((_C,), jnp.int32),
            pltpu.VMEM((2, _C, _D), jnp.float32),
            pltpu.SemaphoreType.DMA((2,)),
            pltpu.SemaphoreType.DMA((2,)),
        ],
        compiler_params=pltpu.CompilerParams(use_tc_tiling_on_sc=False),
    )

    gathered = []
    for t in (loc, pi, spike, slab):
        packed = jnp.reshape(repack(jnp.transpose(t)), (2 * _VP, _D))
        gathered.append(gather(indices, packed))
    g0, g1, g2, g3 = gathered

    nspec = pl.BlockSpec((_TC_BLK, _D), lambda i: (i, 0))
    pdf = pl.pallas_call(
        _pdf_body,
        grid=(_B // _TC_BLK,),
        in_specs=[nspec] * 5,
        out_specs=nspec,
        out_shape=jax.ShapeDtypeStruct((_B, _D), jnp.float32),
    )
    return pdf(X, g0, g1, g2, g3)


def kernel(X, indices, loc, pi, spike, slab):
    return _spike_slab(X, indices.astype(jnp.int32), loc, pi, spike, slab)


# speedup vs baseline: 1.2686x; 1.2686x over previous
import functools

import jax
import jax.numpy as jnp
from jax import lax
from jax.experimental import pallas as pl
from jax.experimental.pallas import tpu as pltpu
from jax.experimental.pallas import tpu_sc as plsc

_B = 16384
_D = 64
_V = 100000
_NC = 2
_NS = 16
_NW = _NC * _NS
_BPW = _B // _NW
_C = 128
_NCHUNK = _BPW // _C
_NT = 4
_L = 16

_RB = 4096
_RBH = _RB // 2
_RB_SHIFT = 12
_NRB = (_V + _RB - 1) // _RB
_VP = _NRB * _RBH
_TC_BLK = 4096

_INV_SQRT_2PI = 0.3989422804014327


def _repack_body(t0_ref, t1_ref, t2_ref, t3_ref,
                 o0_ref, o1_ref, o2_ref, o3_ref):
    eye = jnp.eye(_D, dtype=jnp.float32)
    for t_ref, o_ref in ((t0_ref, o0_ref), (t1_ref, o1_ref),
                         (t2_ref, o2_ref), (t3_ref, o3_ref)):
        t = lax.dot_general(t_ref[...], eye, (((0,), (0,)), ((), ())),
                            preferred_element_type=jnp.float32)
        o_ref[...] = jnp.concatenate([t[:_RBH, :], t[_RBH:, :]], axis=1)


def _gather_body(idx_hbm, t0_hbm, t1_hbm, t2_hbm, t3_hbm,
                 o0_hbm, o1_hbm, o2_hbm, o3_hbm,
                 idx_v, pidx_v0, pidx_v1, g_v, gsems, wsems):
    wid = lax.axis_index("s") * _NC + lax.axis_index("c")
    wbase = wid * _BPW
    pidx_bufs = (pidx_v0, pidx_v1)
    tables = (t0_hbm, t1_hbm, t2_hbm, t3_hbm)
    outs = (o0_hbm, o1_hbm, o2_hbm, o3_hbm)

    def start_gather(c, buf):
        base = wbase + c * _C
        pltpu.sync_copy(idx_hbm.at[pl.ds(base, _C)], idx_v)
        pidx = pidx_bufs[buf]
        for j in range(_C // _L):
            sl = pl.ds(j * _L, _L)
            r = idx_v[sl]
            pidx[sl] = lax.shift_left(
                lax.shift_right_logical(r, _RB_SHIFT), _RB_SHIFT) \
                | lax.shift_left(r & (_RBH - 1), 1) \
                | (lax.shift_right_logical(r, _RB_SHIFT - 1) & 1)
        return [pltpu.async_copy(tables[t].at[pidx], g_v.at[buf, t],
                                 gsems.at[buf, t])
                for t in range(_NT)]

    def start_write(c, buf):
        base = wbase + c * _C
        return [pltpu.async_copy(g_v.at[buf, t], outs[t].at[pl.ds(base, _C)],
                                 wsems.at[buf, t])
                for t in range(_NT)]

    pending = [None, None]
    dmas = start_gather(0, 0)
    for c in range(_NCHUNK):
        buf = c % 2
        nxt = None
        if c + 1 < _NCHUNK:
            nbuf = (c + 1) % 2
            if pending[nbuf] is not None:
                for d in pending[nbuf]:
                    d.wait()
                pending[nbuf] = None
            nxt = start_gather(c + 1, nbuf)
        for d in dmas:
            d.wait()
        pending[buf] = start_write(c, buf)
        dmas = nxt
    for pw in pending:
        if pw is not None:
            for d in pw:
                d.wait()


def _pdf_body(x_ref, lo_ref, p_ref, sp_ref, sb_ref, o_ref):
    x = x_ref[...]
    lo = lo_ref[...]
    p = p_ref[...]
    sp = sp_ref[...]
    sb = sb_ref[...]
    diff = x - lo
    isp = 1.0 / sp
    isb = 1.0 / sb
    zs = diff * isp
    zb = diff * isb
    es = jnp.exp(-0.5 * (zs * zs))
    eb = jnp.exp(-0.5 * (zb * zb))
    o_ref[...] = (p * _INV_SQRT_2PI) * isp * es + \
                 ((1.0 - p) * _INV_SQRT_2PI) * isb * eb


@jax.jit
def _spike_slab(X, indices, loc, pi, spike, slab):
    tspec = pl.BlockSpec((_D, _RB), lambda i: (0, i))
    pspec = pl.BlockSpec((_RBH, 2 * _D), lambda i: (i, 0))
    repack = pl.pallas_call(
        _repack_body,
        grid=(_NRB,),
        in_specs=[tspec] * _NT,
        out_specs=[pspec] * _NT,
        out_shape=[jax.ShapeDtypeStruct((_VP, 2 * _D), jnp.float32)] * _NT,
    )
    packed = repack(jnp.transpose(loc), jnp.transpose(pi),
                    jnp.transpose(spike), jnp.transpose(slab))
    packed = [jnp.reshape(pk, (2 * _VP, _D)) for pk in packed]

    mesh = plsc.VectorSubcoreMesh(core_axis_name="c", subcore_axis_name="s",
                                  num_cores=_NC, num_subcores=_NS)
    gather = pl.kernel(
        _gather_body,
        out_type=[jax.ShapeDtypeStruct((_B, _D), jnp.float32)] * _NT,
        mesh=mesh,
        scratch_types=[
            pltpu.VMEM((_C,), jnp.int32),
            pltpu.VMEM((_C,), jnp.int32),
            pltpu.VMEM((_C,), jnp.int32),
            pltpu.VMEM((2, _NT, _C, _D), jnp.float32),
            pltpu.SemaphoreType.DMA((2, _NT)),
            pltpu.SemaphoreType.DMA((2, _NT)),
        ],
        compiler_params=pltpu.CompilerParams(use_tc_tiling_on_sc=False),
    )
    g0, g1, g2, g3 = gather(indices, *packed)

    nspec = pl.BlockSpec((_TC_BLK, _D), lambda i: (i, 0))
    pdf = pl.pallas_call(
        _pdf_body,
        grid=(_B // _TC_BLK,),
        in_specs=[nspec] * 5,
        out_specs=nspec,
        out_shape=jax.ShapeDtypeStruct((_B, _D), jnp.float32),
    )
    return pdf(X, g0, g1, g2, g3)


def kernel(X, indices, loc, pi, spike, slab):
    return _spike_slab(X, indices.astype(jnp.int32), loc, pi, spike, slab)


# speedup vs baseline: 1.2709x; 1.0018x over previous
import functools

import jax
import jax.numpy as jnp
from jax import lax
from jax.experimental import pallas as pl
from jax.experimental.pallas import tpu as pltpu
from jax.experimental.pallas import tpu_sc as plsc

_B = 16384
_D = 64
_V = 100000
_NC = 2
_NS = 16
_NW = _NC * _NS
_BPW = _B // _NW
_C = 128
_NCHUNK = _BPW // _C
_NT = 4
_L = 16

_RB = 8192
_RBH = _RB // 2
_RB_SHIFT = 13
_NRB = (_V + _RB - 1) // _RB
_VP = _NRB * _RBH
_TC_BLK = 4096

_INV_SQRT_2PI = 0.3989422804014327


def _repack_body(t0_ref, t1_ref, t2_ref, t3_ref,
                 o0_ref, o1_ref, o2_ref, o3_ref):
    eye = jnp.eye(_D, dtype=jnp.float32)
    for t_ref, o_ref in ((t0_ref, o0_ref), (t1_ref, o1_ref),
                         (t2_ref, o2_ref), (t3_ref, o3_ref)):
        t = lax.dot_general(t_ref[...], eye, (((0,), (0,)), ((), ())),
                            preferred_element_type=jnp.float32)
        o_ref[...] = jnp.concatenate([t[:_RBH, :], t[_RBH:, :]], axis=1)


def _gather_body(idx_hbm, t0_hbm, t1_hbm, t2_hbm, t3_hbm,
                 o0_hbm, o1_hbm, o2_hbm, o3_hbm,
                 idx_v, pidx_v0, pidx_v1, g_v, gsems, wsems):
    wid = lax.axis_index("s") * _NC + lax.axis_index("c")
    wbase = wid * _BPW
    pidx_bufs = (pidx_v0, pidx_v1)
    tables = (t0_hbm, t1_hbm, t2_hbm, t3_hbm)
    outs = (o0_hbm, o1_hbm, o2_hbm, o3_hbm)

    def start_gather(c, buf):
        base = wbase + c * _C
        pltpu.sync_copy(idx_hbm.at[pl.ds(base, _C)], idx_v)
        pidx = pidx_bufs[buf]
        for j in range(_C // _L):
            sl = pl.ds(j * _L, _L)
            r = idx_v[sl]
            pidx[sl] = lax.shift_left(
                lax.shift_right_logical(r, _RB_SHIFT), _RB_SHIFT) \
                | lax.shift_left(r & (_RBH - 1), 1) \
                | (lax.shift_right_logical(r, _RB_SHIFT - 1) & 1)
        return [pltpu.async_copy(tables[t].at[pidx], g_v.at[buf, t],
                                 gsems.at[buf, t])
                for t in range(_NT)]

    def start_write(c, buf):
        base = wbase + c * _C
        return [pltpu.async_copy(g_v.at[buf, t], outs[t].at[pl.ds(base, _C)],
                                 wsems.at[buf, t])
                for t in range(_NT)]

    pending = [None, None]
    dmas = start_gather(0, 0)
    for c in range(_NCHUNK):
        buf = c % 2
        nxt = None
        if c + 1 < _NCHUNK:
            nbuf = (c + 1) % 2
            if pending[nbuf] is not None:
                for d in pending[nbuf]:
                    d.wait()
                pending[nbuf] = None
            nxt = start_gather(c + 1, nbuf)
        for d in dmas:
            d.wait()
        pending[buf] = start_write(c, buf)
        dmas = nxt
    for pw in pending:
        if pw is not None:
            for d in pw:
                d.wait()


def _pdf_body(x_ref, lo_ref, p_ref, sp_ref, sb_ref, o_ref):
    x = x_ref[...]
    lo = lo_ref[...]
    p = p_ref[...]
    sp = sp_ref[...]
    sb = sb_ref[...]
    diff = x - lo
    isp = 1.0 / sp
    isb = 1.0 / sb
    zs = diff * isp
    zb = diff * isb
    es = jnp.exp(-0.5 * (zs * zs))
    eb = jnp.exp(-0.5 * (zb * zb))
    o_ref[...] = (p * _INV_SQRT_2PI) * isp * es + \
                 ((1.0 - p) * _INV_SQRT_2PI) * isb * eb


@jax.jit
def _spike_slab(X, indices, loc, pi, spike, slab):
    tspec = pl.BlockSpec((_D, _RB), lambda i: (0, i))
    pspec = pl.BlockSpec((_RBH, 2 * _D), lambda i: (i, 0))
    repack = pl.pallas_call(
        _repack_body,
        grid=(_NRB,),
        in_specs=[tspec] * _NT,
        out_specs=[pspec] * _NT,
        out_shape=[jax.ShapeDtypeStruct((_VP, 2 * _D), jnp.float32)] * _NT,
    )
    packed = repack(jnp.transpose(loc), jnp.transpose(pi),
                    jnp.transpose(spike), jnp.transpose(slab))
    packed = [jnp.reshape(pk, (2 * _VP, _D)) for pk in packed]

    mesh = plsc.VectorSubcoreMesh(core_axis_name="c", subcore_axis_name="s",
                                  num_cores=_NC, num_subcores=_NS)
    gather = pl.kernel(
        _gather_body,
        out_type=[jax.ShapeDtypeStruct((_B, _D), jnp.float32)] * _NT,
        mesh=mesh,
        scratch_types=[
            pltpu.VMEM((_C,), jnp.int32),
            pltpu.VMEM((_C,), jnp.int32),
            pltpu.VMEM((_C,), jnp.int32),
            pltpu.VMEM((2, _NT, _C, _D), jnp.float32),
            pltpu.SemaphoreType.DMA((2, _NT)),
            pltpu.SemaphoreType.DMA((2, _NT)),
        ],
        compiler_params=pltpu.CompilerParams(use_tc_tiling_on_sc=False),
    )
    g0, g1, g2, g3 = gather(indices, *packed)

    nspec = pl.BlockSpec((_TC_BLK, _D), lambda i: (i, 0))
    pdf = pl.pallas_call(
        _pdf_body,
        grid=(_B // _TC_BLK,),
        in_specs=[nspec] * 5,
        out_specs=nspec,
        out_shape=jax.ShapeDtypeStruct((_B, _D), jnp.float32),
    )
    return pdf(X, g0, g1, g2, g3)


def kernel(X, indices, loc, pi, spike, slab):
    return _spike_slab(X, indices.astype(jnp.int32), loc, pi, spike, slab)
